# baseline (speedup 1.0000x reference)
import functools

import jax
import jax.numpy as jnp
from jax import lax
from jax.experimental import pallas as pl
from jax.experimental.pallas import tpu as pltpu

NUM_EXPERTS = 16
TOP_K = 2


def _router_block(x_hbm, w_ref, gates_ref, idx_ref, loss_ref,
                  xbuf, acc_ref, sem, *, block_t, nbuf):
    step = pl.program_id(0)
    nsteps = pl.num_programs(0)
    t = block_t

    def copy_in(src_step, slot):
        return pltpu.make_async_copy(
            x_hbm.at[pl.ds(src_step * t, t), :], xbuf.at[slot], sem.at[slot])

    @pl.when(step == 0)
    def _prime():
        for j in range(nbuf):
            copy_in(j, j).start()

    slot = lax.rem(step, nbuf)
    copy_in(step, slot).wait()

    logits = jax.lax.dot_general(
        w_ref[...], xbuf[slot],
        dimension_numbers=(((1,), (1,)), ((), ())),
        preferred_element_type=jnp.float32,
    )

    @pl.when(step + nbuf < nsteps)
    def _refill():
        copy_in(step + nbuf, slot).start()

    fidx = jax.lax.broadcasted_iota(
        jnp.int32, (NUM_EXPERTS, t), 0).astype(jnp.float32)
    big = jnp.float32(NUM_EXPERTS)

    m1 = jnp.max(logits, axis=0, keepdims=True)
    i1 = jnp.min(jnp.where(logits == m1, fidx, big), axis=0, keepdims=True)

    masked = jnp.where(fidx == i1, -jnp.inf, logits)
    m2 = jnp.max(masked, axis=0, keepdims=True)
    i2 = jnp.min(jnp.where(masked == m2, fidx, big), axis=0, keepdims=True)

    e2 = jnp.exp(m2 - m1)
    g1 = 1.0 / (1.0 + e2)
    g2 = e2 / (1.0 + e2)

    gates = (jnp.where(fidx == i1, g1, 0.0)
             + jnp.where(fidx == i2, g2, 0.0)).astype(jnp.float32)
    gates_ref[...] = gates
    idx_ref[...] = jnp.concatenate([i1, i2], axis=0).astype(jnp.int32)

    part = gates.reshape(NUM_EXPERTS, t // 128, 128).sum(axis=1)

    @pl.when(step == 0)
    def _init():
        acc_ref[...] = part

    @pl.when(step != 0)
    def _acc():
        acc_ref[...] = acc_ref[...] + part

    @pl.when(step == nsteps - 1)
    def _loss():
        total = jnp.float32(t) * nsteps
        usage = jnp.sum(acc_ref[...], axis=1, keepdims=True) / total
        uniform = jnp.float32(1.0 / NUM_EXPERTS)
        kl = jnp.sum(uniform * (jnp.log(uniform) - jnp.log(usage)))
        loss_ref[...] = jnp.full((1, 1), kl, dtype=jnp.float32)


@functools.partial(jax.jit, static_argnames=("block_t", "nbuf"))
def _router(x2d, W, block_t=512, nbuf=6):
    n_tok, d = x2d.shape
    grid = n_tok // block_t
    gates_t, idx_t, loss = pl.pallas_call(
        functools.partial(_router_block, block_t=block_t, nbuf=nbuf),
        grid=(grid,),
        in_specs=[
            pl.BlockSpec(memory_space=pltpu.MemorySpace.HBM),
            pl.BlockSpec((NUM_EXPERTS, d), lambda i: (0, 0)),
        ],
        out_specs=[
            pl.BlockSpec((NUM_EXPERTS, block_t), lambda i: (0, i)),
            pl.BlockSpec((TOP_K, block_t), lambda i: (0, i)),
            pl.BlockSpec((1, 1), lambda i: (0, 0)),
        ],
        out_shape=[
            jax.ShapeDtypeStruct((NUM_EXPERTS, n_tok), jnp.float32),
            jax.ShapeDtypeStruct((TOP_K, n_tok), jnp.int32),
            jax.ShapeDtypeStruct((1, 1), jnp.float32),
        ],
        scratch_shapes=[
            pltpu.VMEM((nbuf, block_t, d), jnp.float32),
            pltpu.VMEM((NUM_EXPERTS, 128), jnp.float32),
            pltpu.SemaphoreType.DMA((nbuf,)),
        ],
    )(x2d, W)
    return gates_t, idx_t, loss


def kernel(x, W):
    b, s, d = x.shape
    x2d = x.reshape(b * s, d)
    gates_t, idx_t, loss = _router(x2d, W)
    return (gates_t.T.reshape(b, s, NUM_EXPERTS),
            idx_t.T.reshape(b, s, TOP_K),
            loss.reshape(()))
